# baseline (speedup 1.0000x reference)
import functools

import jax
import jax.numpy as jnp
from jax import lax
from jax.experimental import pallas as pl
from jax.experimental.pallas import tpu as pltpu
from jax.experimental.pallas import tpu_sc as plsc

_VOCAB = 64
_EMBED = 256
_NC = 2
_NS = 16
_NW = _NC * _NS
_CHUNK = 64
_NBUF = 4
_LOOKAHEAD = 2


def _sc_embed(table, idx_flat):
    B = idx_flat.shape[0]
    b_per_w = B // _NW
    n_chunks = b_per_w // _CHUNK
    mesh = plsc.VectorSubcoreMesh(core_axis_name="c", subcore_axis_name="s")

    @functools.partial(
        pl.kernel,
        mesh=mesh,
        out_type=jax.ShapeDtypeStruct((B, _EMBED), jnp.float32),
        scratch_types=(
            [pltpu.VMEM((b_per_w,), jnp.int32)]
            + [pltpu.VMEM((_CHUNK, _EMBED), jnp.float32) for _ in range(_NBUF)]
            + [pltpu.SemaphoreType.DMA for _ in range(2 * _NBUF)]
        ),
    )
    def k(table_hbm, idx_hbm, out_hbm, idx_v, *bufs_and_sems):
        rows = bufs_and_sems[:_NBUF]
        gsem = bufs_and_sems[_NBUF:2 * _NBUF]
        wsem = bufs_and_sems[2 * _NBUF:]
        wid = lax.axis_index("s") * _NC + lax.axis_index("c")
        base = wid * b_per_w
        pltpu.sync_copy(idx_hbm.at[pl.ds(base, b_per_w)], idx_v)

        def gather(j, b):
            pltpu.async_copy(
                table_hbm.at[idx_v.at[pl.ds(j * _CHUNK, _CHUNK)]],
                rows[b], gsem[b],
            )

        for j in range(_LOOKAHEAD):
            gather(j, j % _NBUF)

        def group(gidx, carry):
            i0 = gidx * _NBUF
            for b in range(_NBUF):
                i = i0 + b
                pltpu.make_async_copy(
                    table_hbm.at[idx_v.at[pl.ds(0, _CHUNK)]], rows[b], gsem[b]
                ).wait()
                pltpu.async_copy(
                    rows[b], out_hbm.at[pl.ds(base + i * _CHUNK, _CHUNK)],
                    wsem[b],
                )
                j = i + _LOOKAHEAD
                bj = (b + _LOOKAHEAD) % _NBUF

                @pl.when(jnp.logical_and(j >= _NBUF, j < n_chunks))
                def _():
                    pltpu.make_async_copy(
                        rows[bj], out_hbm.at[pl.ds(base, _CHUNK)], wsem[bj]
                    ).wait()
                    gather(j, bj)

                @pl.when(jnp.logical_and(j >= _LOOKAHEAD, j < _NBUF))
                def _():
                    gather(j, bj)
            return carry

        lax.fori_loop(0, n_chunks // _NBUF, group, 0)

        for b in range(_NBUF):
            pltpu.make_async_copy(
                rows[b], out_hbm.at[pl.ds(base, _CHUNK)], wsem[b]
            ).wait()

    return k(table, idx_flat)


def kernel(smile_input, embed_weight):
    idx = smile_input.reshape(-1).astype(jnp.int32)
    out = _sc_embed(embed_weight, idx)
    return out.reshape(smile_input.shape + (_EMBED,))


# speedup vs baseline: 1.1345x; 1.1345x over previous
import functools

import jax
import jax.numpy as jnp
from jax import lax
from jax.experimental import pallas as pl
from jax.experimental.pallas import tpu as pltpu
from jax.experimental.pallas import tpu_sc as plsc

_VOCAB = 64
_EMBED = 256
_NC = 2
_NS = 16
_NW = _NC * _NS
_CHUNK = 64


def _sc_embed(table, idx_flat):
    B = idx_flat.shape[0]
    b_per_w = B // _NW
    n_chunks = b_per_w // _CHUNK
    mesh = plsc.VectorSubcoreMesh(core_axis_name="c", subcore_axis_name="s")

    @functools.partial(
        pl.kernel,
        mesh=mesh,
        out_type=jax.ShapeDtypeStruct((B, _EMBED), jnp.float32),
        scratch_types=[
            pltpu.VMEM((b_per_w,), jnp.int32),
            pltpu.VMEM((_VOCAB, _EMBED), jnp.float32),
            pltpu.VMEM((_CHUNK, _EMBED), jnp.float32),
            pltpu.SemaphoreType.DMA,
        ],
    )
    def k(table_hbm, idx_hbm, out_hbm, idx_v, table_v, rows_v, wsem):
        wid = lax.axis_index("s") * _NC + lax.axis_index("c")
        base = wid * b_per_w
        pltpu.sync_copy(table_hbm, table_v)
        pltpu.sync_copy(idx_hbm.at[pl.ds(base, b_per_w)], idx_v)

        def chunk_body(i, carry):
            def group_body(g, carry2):
                gvec = idx_v[pl.ds(i * _CHUNK + g * 16, 16)]
                for l in range(16):
                    ridx = gvec[l]
                    c = g * 16 + l
                    for u in range(_EMBED // 16):
                        rows_v[c, pl.ds(u * 16, 16)] = table_v[
                            ridx, pl.ds(u * 16, 16)
                        ]
                return carry2

            lax.fori_loop(0, _CHUNK // 16, group_body, 0)
            pltpu.async_copy(
                rows_v, out_hbm.at[pl.ds(base + i * _CHUNK, _CHUNK)], wsem
            ).wait()
            return carry

        lax.fori_loop(0, n_chunks, chunk_body, 0)

    return k(table, idx_flat)


def kernel(smile_input, embed_weight):
    idx = smile_input.reshape(-1).astype(jnp.int32)
    out = _sc_embed(embed_weight, idx)
    return out.reshape(smile_input.shape + (_EMBED,))


# speedup vs baseline: 1.5025x; 1.3244x over previous
import functools

import jax
import jax.numpy as jnp
from jax import lax
from jax.experimental import pallas as pl
from jax.experimental.pallas import tpu as pltpu
from jax.experimental.pallas import tpu_sc as plsc

_VOCAB = 64
_EMBED = 256
_NC = 2
_NS = 16
_NW = _NC * _NS
_CHUNK = 64
_NBUF = 2
_LD = 8


def _sc_embed(table, idx_flat):
    B = idx_flat.shape[0]
    b_per_w = B // _NW
    n_chunks = b_per_w // _CHUNK
    mesh = plsc.VectorSubcoreMesh(core_axis_name="c", subcore_axis_name="s")

    @functools.partial(
        pl.kernel,
        mesh=mesh,
        out_type=jax.ShapeDtypeStruct((B, _EMBED), jnp.float32),
        scratch_types=(
            [pltpu.VMEM((b_per_w,), jnp.int32),
             pltpu.VMEM((_VOCAB, _EMBED), jnp.float32)]
            + [pltpu.VMEM((_CHUNK, _EMBED), jnp.float32) for _ in range(_NBUF)]
            + [pltpu.SemaphoreType.DMA for _ in range(_NBUF)]
        ),
    )
    def k(table_hbm, idx_hbm, out_hbm, idx_v, table_v, *bufs_and_sems):
        rows = bufs_and_sems[:_NBUF]
        wsem = bufs_and_sems[_NBUF:]
        wid = lax.axis_index("s") * _NC + lax.axis_index("c")
        base = wid * b_per_w
        pltpu.sync_copy(table_hbm, table_v)
        pltpu.sync_copy(idx_hbm.at[pl.ds(base, b_per_w)], idx_v)

        def fill_and_send(i, b):
            for g in range(_CHUNK // 16):
                gvec = idx_v[pl.ds(i * _CHUNK + g * 16, 16)]
                for l in range(16):
                    ridx = gvec[l]
                    c = g * 16 + l
                    for h in range(_EMBED // (16 * _LD)):
                        vals = [
                            table_v[ridx, pl.ds((h * _LD + u) * 16, 16)]
                            for u in range(_LD)
                        ]
                        for u in range(_LD):
                            rows[b][c, pl.ds((h * _LD + u) * 16, 16)] = vals[u]
            pltpu.async_copy(
                rows[b], out_hbm.at[pl.ds(base + i * _CHUNK, _CHUNK)], wsem[b]
            )

        def wait_write(b):
            pltpu.make_async_copy(
                rows[b], out_hbm.at[pl.ds(base, _CHUNK)], wsem[b]
            ).wait()

        def group(gidx, carry):
            i0 = gidx * _NBUF
            for b in range(_NBUF):
                i = i0 + b

                @pl.when(i >= _NBUF)
                def _():
                    wait_write(b)

                fill_and_send(i, b)
            return carry

        lax.fori_loop(0, n_chunks // _NBUF, group, 0)

        for b in range(_NBUF):
            wait_write(b)

    return k(table, idx_flat)


def kernel(smile_input, embed_weight):
    idx = smile_input.reshape(-1).astype(jnp.int32)
    out = _sc_embed(embed_weight, idx)
    return out.reshape(smile_input.shape + (_EMBED,))
